# baseline (speedup 1.0000x reference)
import functools

import jax
import jax.numpy as jnp
from jax import lax
from jax.experimental import pallas as pl
from jax.experimental.pallas import tpu as pltpu
from jax.experimental.pallas import tpu_sc as plsc

N = 10000
C = 128
R = 64
EPS = 1e-6
K = 27

NC = 2
NS = 16
NW = NC * NS
NP = 10240
RPW = NP // NW
ZROW = 10200

TBL = 262400
TBL_INIT = TBL // NS
SC_DUMP = 262144
G_DUMP = 262300

BLK = 256
NBLK = NP // BLK

_sc_mesh = plsc.VectorSubcoreMesh(core_axis_name="c", subcore_axis_name="s")


@functools.partial(
    pl.kernel,
    out_type=jax.ShapeDtypeStruct((NC * TBL,), jnp.int32),
    mesh=_sc_mesh,
    scratch_types=(
        [pltpu.VMEM((TBL_INIT,), jnp.int32)]
        + [pltpu.VMEM((NP,), jnp.int32) for _ in range(3)]
        + [pltpu.VMEM((64,), jnp.int32) for _ in range(2)]
        + [pltpu.SemaphoreType.DMA]
    ),
)
def _sc_scatter_tbl(ctx, cty, ctz, tbl, zb, ccx, ccy, ccz, lin64, val64, sem):
    cid = lax.axis_index("c")
    sid = lax.axis_index("s")
    base_t = cid * TBL
    lo = sid * TBL_INIT
    hi = lo + TBL_INIT
    dump = SC_DUMP + sid

    @pl.loop(0, TBL_INIT // 16)
    def _(i):
        zb[pl.ds(i * 16, 16)] = jnp.zeros((16,), jnp.int32)

    pltpu.sync_copy(zb, tbl.at[pl.ds(base_t + lo, TBL_INIT)])
    for cc, src_c in ((ccx, ctx), (ccy, cty), (ccz, ctz)):
        pltpu.sync_copy(src_c, cc)

    @pl.loop(0, NP // 64)
    def _(g):
        for u in range(4):
            csl = pl.ds(g * 64 + u * 16, 16)
            x = ccx[csl]
            y = ccy[csl]
            z = ccz[csl]
            ok = (x >= 0) & (x < R) & (y >= 0) & (y < R) & (z >= 0) & (z < R)
            ln = x * (R * R) + y * R + z
            own = ok & (ln >= lo) & (ln < hi)
            ln = jnp.where(own, ln, dump) + base_t
            usl = slice(u * 16, u * 16 + 16)
            lin64[usl] = ln
            val64[usl] = g * 64 + u * 16 + lax.iota(jnp.int32, 16) + 1
        pltpu.async_copy(val64, tbl.at[lin64], sem).wait()


@functools.partial(
    pl.kernel,
    out_type=jax.ShapeDtypeStruct((K * NW * RPW,), jnp.int32),
    mesh=_sc_mesh,
    scratch_types=(
        [pltpu.VMEM((RPW,), jnp.int32) for _ in range(3)]
        + [pltpu.VMEM((64,), jnp.int32) for _ in range(10)]
        + [pltpu.VMEM((RPW,), jnp.int32)]
        + [pltpu.SemaphoreType.DMA]
    ),
)
def _sc_nidx(ctx, cty, ctz, tbl, gidx, *rest):
    cg = rest[0:3]
    nl5 = rest[3:8]
    tv5 = rest[8:13]
    gvv = rest[13]
    sem = rest[14]

    cid = lax.axis_index("c")
    sid = lax.axis_index("s")
    wid = sid * NC + cid
    base_t = cid * TBL
    g0 = wid * RPW
    for d, src_c in enumerate((ctx, cty, ctz)):
        pltpu.sync_copy(src_c.at[pl.ds(g0, RPW)], cg[d])

    @pl.loop(0, K)
    def _(k):
        dx = k // 9 - 1
        dy = (k // 3) % 3 - 1
        dz = k % 3 - 1
        for j in range(20):
            csl = pl.ds(j * 16, 16)
            x = cg[0][csl] + dx
            y = cg[1][csl] + dy
            z = cg[2][csl] + dz
            ok = (x >= 0) & (x < R) & (y >= 0) & (y < R) & (z >= 0) & (z < R)
            ln = x * (R * R) + y * R + z
            ln = jnp.where(ok, ln, G_DUMP) + base_t
            nl5[j // 4][(j % 4) * 16:(j % 4) * 16 + 16] = ln
        ds_ = [pltpu.async_copy(tbl.at[nl5[r]], tv5[r], sem) for r in range(5)]
        for d_ in ds_:
            d_.wait()
        for j in range(20):
            csl = pl.ds(j * 16, 16)
            t = tv5[j // 4][(j % 4) * 16:(j % 4) * 16 + 16]
            gvv[csl] = jnp.where(t > 0, t - 1, ZROW)
        pltpu.sync_copy(gvv, gidx.at[pl.ds((k * NW + wid) * RPW, RPW)])


@functools.partial(
    pl.kernel,
    out_type=jax.ShapeDtypeStruct((NP, K * C), jnp.float32),
    mesh=_sc_mesh,
    scratch_types=(
        [pltpu.VMEM((64,), jnp.int32) for _ in range(5)]
        + [pltpu.VMEM((RPW, C), jnp.float32), pltpu.SemaphoreType.DMA]
    ),
)
def _sc_gather(h, gidx, gout, *rest):
    idx5 = rest[0:5]
    gbuf, sem = rest[5], rest[6]
    cid = lax.axis_index("c")
    sid = lax.axis_index("s")
    wid = sid * NC + cid
    base = wid * RPW

    @pl.loop(0, K)
    def _(k):
        for r in range(5):
            pltpu.sync_copy(
                gidx.at[pl.ds((k * NW + wid) * RPW + r * 64, 64)], idx5[r]
            )
        ds_ = [
            pltpu.async_copy(h.at[idx5[r]], gbuf.at[pl.ds(r * 64, 64)], sem)
            for r in range(5)
        ]
        for d_ in ds_:
            d_.wait()
        pltpu.sync_copy(gbuf, gout.at[pl.ds(base, RPW), pl.ds(k * C, C)])


def _ln_silu(y, rowbase):
    mu = jnp.mean(y, axis=-1, keepdims=True)
    var = jnp.mean((y - mu) ** 2, axis=-1, keepdims=True)
    y = (y - mu) / jnp.sqrt(var + EPS)
    return y


def _mask_rows(y, rowbase):
    rid = rowbase + lax.broadcasted_iota(jnp.int32, (BLK, 1), 0)
    return jnp.where(rid < N, y, 0.0)


def _tc_pre(x_ref, g_ref, b_ref, o_ref):
    rowbase = pl.program_id(0) * BLK
    y = _ln_silu(x_ref[...], rowbase) * g_ref[...] + b_ref[...]
    y = y * jax.nn.sigmoid(y)
    o_ref[...] = _mask_rows(y, rowbase)


def _tc_mid(g_ref, w_ref, b_ref, o_ref):
    rowbase = pl.program_id(0) * BLK
    s = jnp.dot(g_ref[...], w_ref[...], preferred_element_type=jnp.float32)
    s = s + b_ref[...]
    y = _ln_silu(s, rowbase)
    y = y * jax.nn.sigmoid(y)
    o_ref[...] = _mask_rows(y, rowbase)


def _tc_post(g_ref, w_ref, b_ref, f_ref, o_ref):
    s = jnp.dot(g_ref[...], w_ref[...], preferred_element_type=jnp.float32)
    o_ref[...] = s + b_ref[...] + f_ref[...]


_vec_spec = pl.BlockSpec((1, C), lambda i: (0, 0))
_row_spec = pl.BlockSpec((BLK, C), lambda i: (i, 0))
_g_spec = pl.BlockSpec((BLK, K * C), lambda i: (i, 0))
_w_spec = pl.BlockSpec((K * C, C), lambda i: (0, 0))

_pre_call = pl.pallas_call(
    _tc_pre,
    grid=(NBLK,),
    in_specs=[_row_spec, _vec_spec, _vec_spec],
    out_specs=_row_spec,
    out_shape=jax.ShapeDtypeStruct((NP, C), jnp.float32),
)

_mid_call = pl.pallas_call(
    _tc_mid,
    grid=(NBLK,),
    in_specs=[_g_spec, _w_spec, _vec_spec],
    out_specs=_row_spec,
    out_shape=jax.ShapeDtypeStruct((NP, C), jnp.float32),
)

_post_call = pl.pallas_call(
    _tc_post,
    grid=(NBLK,),
    in_specs=[_g_spec, _w_spec, _vec_spec, _row_spec],
    out_specs=_row_spec,
    out_shape=jax.ShapeDtypeStruct((NP, C), jnp.float32),
)


def kernel(feats, coords, gamma, beta, W1, b1, W2, b2):
    f32 = jnp.float32
    fp = jnp.zeros((NP, C), f32).at[:N].set(feats.astype(f32))
    cp = jnp.full((NP, 3), R, jnp.int32).at[:N].set(coords.astype(jnp.int32))

    cx, cy, cz = cp[:, 0].copy(), cp[:, 1].copy(), cp[:, 2].copy()
    tbl = _sc_scatter_tbl(cx, cy, cz)
    gidx = _sc_nidx(cx, cy, cz, tbl)

    h1 = _pre_call(fp, gamma.reshape(1, C), beta.reshape(1, C))
    G1 = _sc_gather(h1, gidx)
    h2 = _mid_call(G1, W1.reshape(K * C, C), b1.reshape(1, C))
    G2 = _sc_gather(h2, gidx)
    out = _post_call(G2, W2.reshape(K * C, C), b2.reshape(1, C), fp)
    return out[:N]


# speedup vs baseline: 2.1565x; 2.1565x over previous
import functools

import jax
import jax.numpy as jnp
from jax import lax
from jax.experimental import pallas as pl
from jax.experimental.pallas import tpu as pltpu
from jax.experimental.pallas import tpu_sc as plsc

N = 10000
C = 128
R = 64
EPS = 1e-6
K = 27

NC = 2
NS = 16
NW = NC * NS
NP = 10240
RPW = NP // NW
ZROW = 10200

TBL = 262400
TBL_INIT = TBL // NS
SC_DUMP = 262144
G_DUMP = 262300

BLK = 256
NBLK = NP // BLK

_sc_mesh = plsc.VectorSubcoreMesh(core_axis_name="c", subcore_axis_name="s")


@functools.partial(
    pl.kernel,
    out_type=jax.ShapeDtypeStruct((NW * K * RPW,), jnp.int32),
    mesh=_sc_mesh,
    scratch_types=(
        [pltpu.VMEM_SHARED((TBL,), jnp.int32)]
        + [pltpu.VMEM((TBL_INIT,), jnp.int32)]
        + [pltpu.VMEM((NP,), jnp.int32) for _ in range(3)]
        + [pltpu.VMEM((128,), jnp.int32) for _ in range(2)]
        + [pltpu.VMEM((RPW,), jnp.int32) for _ in range(3)]
        + [pltpu.VMEM((64,), jnp.int32) for _ in range(10)]
        + [pltpu.VMEM((RPW,), jnp.int32)]
        + [pltpu.SemaphoreType.DMA]
    ),
)
def _sc_index(ctx, cty, ctz, gidx, shr, zb, *rest):
    ccx, ccy, ccz = rest[0:3]
    lin128, val128 = rest[3:5]
    cg = rest[5:8]
    nl5 = rest[8:13]
    tv5 = rest[13:18]
    gvv = rest[18]
    sem = rest[19]

    cid = lax.axis_index("c")
    sid = lax.axis_index("s")
    wid = sid * NC + cid
    lo = sid * TBL_INIT
    hi = lo + TBL_INIT
    dump = SC_DUMP + sid

    @pl.loop(0, TBL_INIT // 16)
    def _(i):
        zb[pl.ds(i * 16, 16)] = jnp.zeros((16,), jnp.int32)

    pltpu.sync_copy(zb, shr.at[pl.ds(lo, TBL_INIT)])

    for cc, src_c in ((ccx, ctx), (ccy, cty), (ccz, ctz)):
        pltpu.sync_copy(src_c, cc)

    @pl.loop(0, NP // 128)
    def _(g):
        for u in range(8):
            csl = pl.ds(g * 128 + u * 16, 16)
            x = ccx[csl]
            y = ccy[csl]
            z = ccz[csl]
            ok = (x >= 0) & (x < R) & (y >= 0) & (y < R) & (z >= 0) & (z < R)
            ln = x * (R * R) + y * R + z
            own = ok & (ln >= lo) & (ln < hi)
            ln = jnp.where(own, ln, dump)
            usl = slice(u * 16, u * 16 + 16)
            lin128[usl] = ln
            val128[usl] = g * 128 + u * 16 + lax.iota(jnp.int32, 16) + 1
        pltpu.async_copy(val128, shr.at[lin128], sem).wait()

    plsc.subcore_barrier()

    g0 = wid * RPW
    for d, src_c in enumerate((ctx, cty, ctz)):
        pltpu.sync_copy(src_c.at[pl.ds(g0, RPW)], cg[d])

    @pl.loop(0, K)
    def _(k):
        dx = k // 9 - 1
        dy = (k // 3) % 3 - 1
        dz = k % 3 - 1
        for j in range(20):
            csl = pl.ds(j * 16, 16)
            x = cg[0][csl] + dx
            y = cg[1][csl] + dy
            z = cg[2][csl] + dz
            ok = (x >= 0) & (x < R) & (y >= 0) & (y < R) & (z >= 0) & (z < R)
            ln = x * (R * R) + y * R + z
            ln = jnp.where(ok, ln, G_DUMP)
            nl5[j // 4][(j % 4) * 16:(j % 4) * 16 + 16] = ln
        ds_ = [pltpu.async_copy(shr.at[nl5[r]], tv5[r], sem) for r in range(5)]
        for d_ in ds_:
            d_.wait()
        for j in range(20):
            csl = pl.ds(j * 16, 16)
            t = tv5[j // 4][(j % 4) * 16:(j % 4) * 16 + 16]
            gvv[csl] = jnp.where(t > 0, t - 1, ZROW)
        pltpu.sync_copy(gvv, gidx.at[pl.ds((wid * K + k) * RPW, RPW)])


@functools.partial(
    pl.kernel,
    out_type=jax.ShapeDtypeStruct((K * NP, C), jnp.float32),
    mesh=_sc_mesh,
    scratch_types=[
        pltpu.VMEM((K * RPW,), jnp.int32),
        pltpu.VMEM((RPW, C), jnp.float32),
        pltpu.VMEM((RPW, C), jnp.float32),
        pltpu.SemaphoreType.DMA,
        pltpu.SemaphoreType.DMA,
    ],
)
def _sc_gather(h, gidx, gout, idxall, b0, b1, gsem, ssem):
    cid = lax.axis_index("c")
    sid = lax.axis_index("s")
    wid = sid * NC + cid
    base = wid * RPW
    bufs = (b0, b1)
    pltpu.sync_copy(gidx.at[pl.ds(wid * (K * RPW), K * RPW)], idxall)
    store_descs = [None, None]
    gath_descs = [None, None]
    for k in range(K):
        b = k % 2
        if store_descs[b] is not None:
            store_descs[b].wait()
        gath_descs[b] = [
            pltpu.async_copy(
                h.at[idxall.at[pl.ds(k * RPW + r * 64, 64)]],
                bufs[b].at[pl.ds(r * 64, 64)],
                gsem,
            )
            for r in range(5)
        ]
        pb = (k + 1) % 2
        if gath_descs[pb] is not None:
            for d_ in gath_descs[pb]:
                d_.wait()
            store_descs[pb] = pltpu.async_copy(
                bufs[pb], gout.at[pl.ds((k - 1) * NP + base, RPW)], ssem
            )
            gath_descs[pb] = None
    for d_ in gath_descs[(K - 1) % 2]:
        d_.wait()
    store_descs[(K - 1) % 2] = pltpu.async_copy(
        bufs[(K - 1) % 2], gout.at[pl.ds((K - 1) * NP + base, RPW)], ssem
    )
    for d_ in store_descs:
        if d_ is not None:
            d_.wait()


def _ln(y):
    mu = jnp.mean(y, axis=-1, keepdims=True)
    var = jnp.mean((y - mu) ** 2, axis=-1, keepdims=True)
    return (y - mu) / jnp.sqrt(var + EPS)


def _mask_rows(y, rowbase):
    rid = rowbase + lax.broadcasted_iota(jnp.int32, (BLK, 1), 0)
    return jnp.where(rid < N, y, 0.0)


def _conv_acc(g_ref, w_ref):
    acc = jnp.zeros((BLK, C), jnp.float32)
    for k in range(K):
        acc = acc + jnp.dot(g_ref[k], w_ref[k], preferred_element_type=jnp.float32)
    return acc


def _tc_pre(x_ref, g_ref, b_ref, o_ref):
    rowbase = pl.program_id(0) * BLK
    y = _ln(x_ref[...]) * g_ref[...] + b_ref[...]
    y = y * jax.nn.sigmoid(y)
    o_ref[...] = _mask_rows(y, rowbase)


def _tc_mid(g_ref, w_ref, b_ref, o_ref):
    rowbase = pl.program_id(0) * BLK
    y = _ln(_conv_acc(g_ref, w_ref) + b_ref[...])
    y = y * jax.nn.sigmoid(y)
    o_ref[...] = _mask_rows(y, rowbase)


def _tc_post(g_ref, w_ref, b_ref, f_ref, o_ref):
    o_ref[...] = _conv_acc(g_ref, w_ref) + b_ref[...] + f_ref[...]


_vec_spec = pl.BlockSpec((1, C), lambda i: (0, 0))
_row_spec = pl.BlockSpec((BLK, C), lambda i: (i, 0))
_g_spec = pl.BlockSpec((K, BLK, C), lambda i: (0, i, 0))
_w_spec = pl.BlockSpec((K, C, C), lambda i: (0, 0, 0))

_pre_call = pl.pallas_call(
    _tc_pre,
    grid=(NBLK,),
    in_specs=[_row_spec, _vec_spec, _vec_spec],
    out_specs=_row_spec,
    out_shape=jax.ShapeDtypeStruct((NP, C), jnp.float32),
)

_mid_call = pl.pallas_call(
    _tc_mid,
    grid=(NBLK,),
    in_specs=[_g_spec, _w_spec, _vec_spec],
    out_specs=_row_spec,
    out_shape=jax.ShapeDtypeStruct((NP, C), jnp.float32),
)

_post_call = pl.pallas_call(
    _tc_post,
    grid=(NBLK,),
    in_specs=[_g_spec, _w_spec, _vec_spec, _row_spec],
    out_specs=_row_spec,
    out_shape=jax.ShapeDtypeStruct((NP, C), jnp.float32),
)


def kernel(feats, coords, gamma, beta, W1, b1, W2, b2):
    f32 = jnp.float32
    fp = jnp.zeros((NP, C), f32).at[:N].set(feats.astype(f32))
    cp = jnp.full((NP, 3), R, jnp.int32).at[:N].set(coords.astype(jnp.int32))
    cx, cy, cz = cp[:, 0].copy(), cp[:, 1].copy(), cp[:, 2].copy()

    gidx = _sc_index(cx, cy, cz)

    h1 = _pre_call(fp, gamma.reshape(1, C), beta.reshape(1, C))
    G1 = _sc_gather(h1, gidx).reshape(K, NP, C)
    h2 = _mid_call(G1, W1, b1.reshape(1, C))
    G2 = _sc_gather(h2, gidx).reshape(K, NP, C)
    out = _post_call(G2, W2, b2.reshape(1, C), fp)
    return out[:N]


# speedup vs baseline: 47.3720x; 21.9675x over previous
import functools

import jax
import jax.numpy as jnp
from jax import lax
from jax.experimental import pallas as pl
from jax.experimental.pallas import tpu as pltpu
from jax.experimental.pallas import tpu_sc as plsc

N = 10000
C = 128
R = 64
EPS = 1e-6
K = 27

NC = 2
NS = 16
NW = NC * NS
NP = 10240
RPW = NP // NW
ZROW = 10200

TBL = 262400
TBL_INIT = TBL // NS
SC_DUMP = 262144
G_DUMP = 262300

BLK = 256
NBLK = NP // BLK

_sc_mesh = plsc.VectorSubcoreMesh(core_axis_name="c", subcore_axis_name="s")


@functools.partial(
    pl.kernel,
    out_type=jax.ShapeDtypeStruct((NW * K * RPW,), jnp.int32),
    mesh=_sc_mesh,
    scratch_types=(
        [pltpu.VMEM_SHARED((TBL,), jnp.int32)]
        + [pltpu.VMEM((TBL_INIT,), jnp.int32)]
        + [pltpu.VMEM((NP,), jnp.int32) for _ in range(3)]
        + [pltpu.VMEM((128,), jnp.int32) for _ in range(2)]
        + [pltpu.VMEM((RPW,), jnp.int32) for _ in range(3)]
        + [pltpu.VMEM((64,), jnp.int32) for _ in range(10)]
        + [pltpu.VMEM((RPW,), jnp.int32)]
        + [pltpu.SemaphoreType.DMA]
    ),
)
def _sc_index(ctx, cty, ctz, gidx, shr, zb, *rest):
    ccx, ccy, ccz = rest[0:3]
    lin128, val128 = rest[3:5]
    cg = rest[5:8]
    nl5 = rest[8:13]
    tv5 = rest[13:18]
    gvv = rest[18]
    sem = rest[19]

    cid = lax.axis_index("c")
    sid = lax.axis_index("s")
    wid = sid * NC + cid
    lo = sid * TBL_INIT
    hi = lo + TBL_INIT
    dump = SC_DUMP + sid

    @pl.loop(0, TBL_INIT // 16)
    def _(i):
        zb[pl.ds(i * 16, 16)] = jnp.zeros((16,), jnp.int32)

    pltpu.sync_copy(zb, shr.at[pl.ds(lo, TBL_INIT)])

    for cc, src_c in ((ccx, ctx), (ccy, cty), (ccz, ctz)):
        pltpu.sync_copy(src_c, cc)

    @pl.loop(0, NP // 128)
    def _(g):
        for u in range(8):
            csl = pl.ds(g * 128 + u * 16, 16)
            x = ccx[csl]
            y = ccy[csl]
            z = ccz[csl]
            ok = (x >= 0) & (x < R) & (y >= 0) & (y < R) & (z >= 0) & (z < R)
            ln = x * (R * R) + y * R + z
            own = ok & (ln >= lo) & (ln < hi)
            ln = jnp.where(own, ln, dump)
            usl = slice(u * 16, u * 16 + 16)
            lin128[usl] = ln
            val128[usl] = g * 128 + u * 16 + lax.iota(jnp.int32, 16) + 1
        pltpu.async_copy(val128, shr.at[lin128], sem).wait()

    plsc.subcore_barrier()

    g0 = wid * RPW
    for d, src_c in enumerate((ctx, cty, ctz)):
        pltpu.sync_copy(src_c.at[pl.ds(g0, RPW)], cg[d])

    @pl.loop(0, K)
    def _(k):
        dx = k // 9 - 1
        dy = (k // 3) % 3 - 1
        dz = k % 3 - 1
        for j in range(20):
            csl = pl.ds(j * 16, 16)
            x = cg[0][csl] + dx
            y = cg[1][csl] + dy
            z = cg[2][csl] + dz
            ok = (x >= 0) & (x < R) & (y >= 0) & (y < R) & (z >= 0) & (z < R)
            ln = x * (R * R) + y * R + z
            ln = jnp.where(ok, ln, G_DUMP)
            nl5[j // 4][(j % 4) * 16:(j % 4) * 16 + 16] = ln
        ds_ = [pltpu.async_copy(shr.at[nl5[r]], tv5[r], sem) for r in range(5)]
        for d_ in ds_:
            d_.wait()
        for j in range(20):
            csl = pl.ds(j * 16, 16)
            t = tv5[j // 4][(j % 4) * 16:(j % 4) * 16 + 16]
            zr = N + ((wid * 67 + k * 37 + j * 16 + lax.iota(jnp.int32, 16)) & 127)
            gvv[csl] = jnp.where(t > 0, t - 1, zr)
        pltpu.sync_copy(gvv, gidx.at[pl.ds((wid * K + k) * RPW, RPW)])


@functools.partial(
    pl.kernel,
    out_type=jax.ShapeDtypeStruct((K * NP, C), jnp.float32),
    mesh=_sc_mesh,
    scratch_types=[
        pltpu.VMEM((K * RPW,), jnp.int32),
        pltpu.VMEM((RPW, C), jnp.float32),
        pltpu.VMEM((RPW, C), jnp.float32),
        pltpu.SemaphoreType.DMA,
        pltpu.SemaphoreType.DMA,
    ],
)
def _sc_gather(h, gidx, gout, idxall, b0, b1, gsem, ssem):
    cid = lax.axis_index("c")
    sid = lax.axis_index("s")
    wid = sid * NC + cid
    base = wid * RPW
    bufs = (b0, b1)
    pltpu.sync_copy(gidx.at[pl.ds(wid * (K * RPW), K * RPW)], idxall)
    store_descs = [None, None]
    gath_descs = [None, None]
    for k in range(K):
        b = k % 2
        if store_descs[b] is not None:
            store_descs[b].wait()
        gath_descs[b] = [
            pltpu.async_copy(
                h.at[idxall.at[pl.ds(k * RPW + r * 64, 64)]],
                bufs[b].at[pl.ds(r * 64, 64)],
                gsem,
            )
            for r in range(5)
        ]
        pb = (k + 1) % 2
        if gath_descs[pb] is not None:
            for d_ in gath_descs[pb]:
                d_.wait()
            store_descs[pb] = pltpu.async_copy(
                bufs[pb], gout.at[pl.ds((k - 1) * NP + base, RPW)], ssem
            )
            gath_descs[pb] = None
    for d_ in gath_descs[(K - 1) % 2]:
        d_.wait()
    store_descs[(K - 1) % 2] = pltpu.async_copy(
        bufs[(K - 1) % 2], gout.at[pl.ds((K - 1) * NP + base, RPW)], ssem
    )
    for d_ in store_descs:
        if d_ is not None:
            d_.wait()


def _ln(y):
    mu = jnp.mean(y, axis=-1, keepdims=True)
    var = jnp.mean((y - mu) ** 2, axis=-1, keepdims=True)
    return (y - mu) / jnp.sqrt(var + EPS)


def _mask_rows(y, rowbase):
    rid = rowbase + lax.broadcasted_iota(jnp.int32, (BLK, 1), 0)
    return jnp.where(rid < N, y, 0.0)


def _conv_acc(g_ref, w_ref):
    acc = jnp.zeros((BLK, C), jnp.float32)
    for k in range(K):
        acc = acc + jnp.dot(g_ref[k], w_ref[k], preferred_element_type=jnp.float32)
    return acc


def _tc_pre(x_ref, g_ref, b_ref, o_ref):
    rowbase = pl.program_id(0) * BLK
    y = _ln(x_ref[...]) * g_ref[...] + b_ref[...]
    y = y * jax.nn.sigmoid(y)
    o_ref[...] = _mask_rows(y, rowbase)


def _tc_mid(g_ref, w_ref, b_ref, o_ref):
    rowbase = pl.program_id(0) * BLK
    y = _ln(_conv_acc(g_ref, w_ref) + b_ref[...])
    y = y * jax.nn.sigmoid(y)
    o_ref[...] = _mask_rows(y, rowbase)


def _tc_post(g_ref, w_ref, b_ref, f_ref, o_ref):
    o_ref[...] = _conv_acc(g_ref, w_ref) + b_ref[...] + f_ref[...]


_vec_spec = pl.BlockSpec((1, C), lambda i: (0, 0))
_row_spec = pl.BlockSpec((BLK, C), lambda i: (i, 0))
_g_spec = pl.BlockSpec((K, BLK, C), lambda i: (0, i, 0))
_w_spec = pl.BlockSpec((K, C, C), lambda i: (0, 0, 0))

_pre_call = pl.pallas_call(
    _tc_pre,
    grid=(NBLK,),
    in_specs=[_row_spec, _vec_spec, _vec_spec],
    out_specs=_row_spec,
    out_shape=jax.ShapeDtypeStruct((NP, C), jnp.float32),
)

_mid_call = pl.pallas_call(
    _tc_mid,
    grid=(NBLK,),
    in_specs=[_g_spec, _w_spec, _vec_spec],
    out_specs=_row_spec,
    out_shape=jax.ShapeDtypeStruct((NP, C), jnp.float32),
)

_post_call = pl.pallas_call(
    _tc_post,
    grid=(NBLK,),
    in_specs=[_g_spec, _w_spec, _vec_spec, _row_spec],
    out_specs=_row_spec,
    out_shape=jax.ShapeDtypeStruct((NP, C), jnp.float32),
)


def kernel(feats, coords, gamma, beta, W1, b1, W2, b2):
    f32 = jnp.float32
    fp = jnp.zeros((NP, C), f32).at[:N].set(feats.astype(f32))
    cp = jnp.full((NP, 3), R, jnp.int32).at[:N].set(coords.astype(jnp.int32))
    cx, cy, cz = cp[:, 0].copy(), cp[:, 1].copy(), cp[:, 2].copy()

    gidx = _sc_index(cx, cy, cz)

    h1 = _pre_call(fp, gamma.reshape(1, C), beta.reshape(1, C))
    G1 = _sc_gather(h1, gidx).reshape(K, NP, C)
    h2 = _mid_call(G1, W1, b1.reshape(1, C))
    G2 = _sc_gather(h2, gidx).reshape(K, NP, C)
    out = _post_call(G2, W2, b2.reshape(1, C), fp)
    return out[:N]


# speedup vs baseline: 47.6918x; 1.0068x over previous
import functools

import jax
import jax.numpy as jnp
from jax import lax
from jax.experimental import pallas as pl
from jax.experimental.pallas import tpu as pltpu
from jax.experimental.pallas import tpu_sc as plsc

N = 10000
C = 128
R = 64
EPS = 1e-6
K = 27

NC = 2
NS = 16
NW = NC * NS
NP = 10240
RPW = NP // NW
ZROW = 10200

TBL = 262400
TBL_INIT = TBL // NS
SC_DUMP = 262144
G_DUMP = 262300

BLK = 256
NBLK = NP // BLK

_sc_mesh = plsc.VectorSubcoreMesh(core_axis_name="c", subcore_axis_name="s")


@functools.partial(
    pl.kernel,
    out_type=jax.ShapeDtypeStruct((NW * K * RPW,), jnp.int32),
    mesh=_sc_mesh,
    scratch_types=(
        [pltpu.VMEM_SHARED((TBL,), jnp.int32)]
        + [pltpu.VMEM((TBL_INIT,), jnp.int32)]
        + [pltpu.VMEM((NP,), jnp.int32) for _ in range(3)]
        + [pltpu.VMEM((128,), jnp.int32) for _ in range(2)]
        + [pltpu.VMEM((RPW,), jnp.int32) for _ in range(3)]
        + [pltpu.VMEM((64,), jnp.int32) for _ in range(10)]
        + [pltpu.VMEM((RPW,), jnp.int32)]
        + [pltpu.SemaphoreType.DMA]
    ),
)
def _sc_index(ctx, cty, ctz, gidx, shr, zb, *rest):
    ccx, ccy, ccz = rest[0:3]
    lin128, val128 = rest[3:5]
    cg = rest[5:8]
    nl5 = rest[8:13]
    tv5 = rest[13:18]
    gvv = rest[18]
    sem = rest[19]

    cid = lax.axis_index("c")
    sid = lax.axis_index("s")
    wid = sid * NC + cid
    lo = sid * TBL_INIT
    hi = lo + TBL_INIT
    dump = SC_DUMP + sid

    @pl.loop(0, TBL_INIT // 16)
    def _(i):
        zb[pl.ds(i * 16, 16)] = jnp.zeros((16,), jnp.int32)

    pltpu.sync_copy(zb, shr.at[pl.ds(lo, TBL_INIT)])

    for cc, src_c in ((ccx, ctx), (ccy, cty), (ccz, ctz)):
        pltpu.sync_copy(src_c, cc)

    @pl.loop(0, NP // 128)
    def _(g):
        for u in range(8):
            csl = pl.ds(g * 128 + u * 16, 16)
            x = ccx[csl]
            y = ccy[csl]
            z = ccz[csl]
            ok = (x >= 0) & (x < R) & (y >= 0) & (y < R) & (z >= 0) & (z < R)
            ln = x * (R * R) + y * R + z
            own = ok & (ln >= lo) & (ln < hi)
            ln = jnp.where(own, ln, dump)
            usl = slice(u * 16, u * 16 + 16)
            lin128[usl] = ln
            val128[usl] = g * 128 + u * 16 + lax.iota(jnp.int32, 16) + 1
        pltpu.async_copy(val128, shr.at[lin128], sem).wait()

    plsc.subcore_barrier()

    g0 = wid * RPW
    for d, src_c in enumerate((ctx, cty, ctz)):
        pltpu.sync_copy(src_c.at[pl.ds(g0, RPW)], cg[d])

    @pl.loop(0, K)
    def _(k):
        dx = k // 9 - 1
        dy = (k // 3) % 3 - 1
        dz = k % 3 - 1
        for j in range(20):
            csl = pl.ds(j * 16, 16)
            x = cg[0][csl] + dx
            y = cg[1][csl] + dy
            z = cg[2][csl] + dz
            ok = (x >= 0) & (x < R) & (y >= 0) & (y < R) & (z >= 0) & (z < R)
            ln = x * (R * R) + y * R + z
            ln = jnp.where(ok, ln, G_DUMP)
            nl5[j // 4][(j % 4) * 16:(j % 4) * 16 + 16] = ln
        ds_ = [pltpu.async_copy(shr.at[nl5[r]], tv5[r], sem) for r in range(5)]
        for d_ in ds_:
            d_.wait()
        for j in range(20):
            csl = pl.ds(j * 16, 16)
            t = tv5[j // 4][(j % 4) * 16:(j % 4) * 16 + 16]
            zr = N + ((wid * 67 + k * 37 + j * 16 + lax.iota(jnp.int32, 16)) & 127)
            gvv[csl] = jnp.where(t > 0, t - 1, zr)
        pltpu.sync_copy(gvv, gidx.at[pl.ds((wid * K + k) * RPW, RPW)])


@functools.partial(
    pl.kernel,
    out_type=jax.ShapeDtypeStruct((K * NP, C), jnp.float32),
    mesh=_sc_mesh,
    scratch_types=(
        [pltpu.VMEM((K * RPW,), jnp.int32)]
        + [pltpu.VMEM((RPW, C), jnp.float32) for _ in range(2)]
        + [pltpu.SemaphoreType.DMA for _ in range(20)]
    ),
)
def _sc_gather(h, gidx, gout, idxall, b0, b1, *sems):
    gsems = (sems[0:5], sems[5:10])
    ssems = (sems[10:15], sems[15:20])
    cid = lax.axis_index("c")
    sid = lax.axis_index("s")
    wid = sid * NC + cid
    base = wid * RPW
    bufs = (b0, b1)
    pltpu.sync_copy(gidx.at[pl.ds(wid * (K * RPW), K * RPW)], idxall)
    gath = [[None] * 5, [None] * 5]
    stor = [[None] * 5, [None] * 5]
    for k in range(K + 1):
        b = k % 2
        if k < K:
            for r in range(5):
                if stor[b][r] is not None:
                    stor[b][r].wait()
                gath[b][r] = pltpu.async_copy(
                    h.at[idxall.at[pl.ds(k * RPW + r * 64, 64)]],
                    bufs[b].at[pl.ds(r * 64, 64)],
                    gsems[b][r],
                )
        pb = (k + 1) % 2
        if k >= 1:
            for r in range(5):
                gath[pb][r].wait()
                stor[pb][r] = pltpu.async_copy(
                    bufs[pb].at[pl.ds(r * 64, 64)],
                    gout.at[pl.ds((k - 1) * NP + base + r * 64, 64)],
                    ssems[pb][r],
                )
    for r in range(5):
        stor[(K - 1) % 2][r].wait()


def _ln(y):
    mu = jnp.mean(y, axis=-1, keepdims=True)
    var = jnp.mean((y - mu) ** 2, axis=-1, keepdims=True)
    return (y - mu) / jnp.sqrt(var + EPS)


def _mask_rows(y, rowbase):
    rid = rowbase + lax.broadcasted_iota(jnp.int32, (BLK, 1), 0)
    return jnp.where(rid < N, y, 0.0)


def _conv_acc(g_ref, w_ref):
    acc = jnp.zeros((BLK, C), jnp.float32)
    for k in range(K):
        acc = acc + jnp.dot(g_ref[k], w_ref[k], preferred_element_type=jnp.float32)
    return acc


def _tc_pre(x_ref, g_ref, b_ref, o_ref):
    rowbase = pl.program_id(0) * BLK
    y = _ln(x_ref[...]) * g_ref[...] + b_ref[...]
    y = y * jax.nn.sigmoid(y)
    o_ref[...] = _mask_rows(y, rowbase)


def _tc_mid(g_ref, w_ref, b_ref, o_ref):
    rowbase = pl.program_id(0) * BLK
    y = _ln(_conv_acc(g_ref, w_ref) + b_ref[...])
    y = y * jax.nn.sigmoid(y)
    o_ref[...] = _mask_rows(y, rowbase)


def _tc_post(g_ref, w_ref, b_ref, f_ref, o_ref):
    o_ref[...] = _conv_acc(g_ref, w_ref) + b_ref[...] + f_ref[...]


_vec_spec = pl.BlockSpec((1, C), lambda i: (0, 0))
_row_spec = pl.BlockSpec((BLK, C), lambda i: (i, 0))
_g_spec = pl.BlockSpec((K, BLK, C), lambda i: (0, i, 0))
_w_spec = pl.BlockSpec((K, C, C), lambda i: (0, 0, 0))

_pre_call = pl.pallas_call(
    _tc_pre,
    grid=(NBLK,),
    in_specs=[_row_spec, _vec_spec, _vec_spec],
    out_specs=_row_spec,
    out_shape=jax.ShapeDtypeStruct((NP, C), jnp.float32),
)

_mid_call = pl.pallas_call(
    _tc_mid,
    grid=(NBLK,),
    in_specs=[_g_spec, _w_spec, _vec_spec],
    out_specs=_row_spec,
    out_shape=jax.ShapeDtypeStruct((NP, C), jnp.float32),
)

_post_call = pl.pallas_call(
    _tc_post,
    grid=(NBLK,),
    in_specs=[_g_spec, _w_spec, _vec_spec, _row_spec],
    out_specs=_row_spec,
    out_shape=jax.ShapeDtypeStruct((NP, C), jnp.float32),
)


def kernel(feats, coords, gamma, beta, W1, b1, W2, b2):
    f32 = jnp.float32
    fp = jnp.zeros((NP, C), f32).at[:N].set(feats.astype(f32))
    cp = jnp.full((NP, 3), R, jnp.int32).at[:N].set(coords.astype(jnp.int32))
    cx, cy, cz = cp[:, 0].copy(), cp[:, 1].copy(), cp[:, 2].copy()

    gidx = _sc_index(cx, cy, cz)

    h1 = _pre_call(fp, gamma.reshape(1, C), beta.reshape(1, C))
    G1 = _sc_gather(h1, gidx).reshape(K, NP, C)
    h2 = _mid_call(G1, W1, b1.reshape(1, C))
    G2 = _sc_gather(h2, gidx).reshape(K, NP, C)
    out = _post_call(G2, W2, b2.reshape(1, C), fp)
    return out[:N]


# speedup vs baseline: 48.8676x; 1.0247x over previous
import functools

import jax
import jax.numpy as jnp
from jax import lax
from jax.experimental import pallas as pl
from jax.experimental.pallas import tpu as pltpu
from jax.experimental.pallas import tpu_sc as plsc

N = 10000
C = 128
R = 64
EPS = 1e-6
K = 27

NC = 2
NS = 16
NW = NC * NS
NP = 10240
RPW = NP // NW
ZROW = 10200

TBL = 262400
TBL_INIT = TBL // NS
SC_DUMP = 262144
G_DUMP = 262300

BLK = 512
NBLK = NP // BLK

_sc_mesh = plsc.VectorSubcoreMesh(core_axis_name="c", subcore_axis_name="s")


@functools.partial(
    pl.kernel,
    out_type=jax.ShapeDtypeStruct((NW * K * RPW,), jnp.int32),
    mesh=_sc_mesh,
    scratch_types=(
        [pltpu.VMEM_SHARED((TBL,), jnp.int32)]
        + [pltpu.VMEM((TBL_INIT,), jnp.int32)]
        + [pltpu.VMEM((NP,), jnp.int32) for _ in range(3)]
        + [pltpu.VMEM((128,), jnp.int32) for _ in range(2)]
        + [pltpu.VMEM((RPW,), jnp.int32) for _ in range(3)]
        + [pltpu.VMEM((64,), jnp.int32) for _ in range(10)]
        + [pltpu.VMEM((RPW,), jnp.int32)]
        + [pltpu.SemaphoreType.DMA]
    ),
)
def _sc_index(ctx, cty, ctz, gidx, shr, zb, *rest):
    ccx, ccy, ccz = rest[0:3]
    lin128, val128 = rest[3:5]
    cg = rest[5:8]
    nl5 = rest[8:13]
    tv5 = rest[13:18]
    gvv = rest[18]
    sem = rest[19]

    cid = lax.axis_index("c")
    sid = lax.axis_index("s")
    wid = sid * NC + cid
    lo = sid * TBL_INIT
    hi = lo + TBL_INIT
    dump = SC_DUMP + sid

    @pl.loop(0, TBL_INIT // 16)
    def _(i):
        zb[pl.ds(i * 16, 16)] = jnp.zeros((16,), jnp.int32)

    pltpu.sync_copy(zb, shr.at[pl.ds(lo, TBL_INIT)])

    for cc, src_c in ((ccx, ctx), (ccy, cty), (ccz, ctz)):
        pltpu.sync_copy(src_c, cc)

    @pl.loop(0, NP // 128)
    def _(g):
        for u in range(8):
            csl = pl.ds(g * 128 + u * 16, 16)
            x = ccx[csl]
            y = ccy[csl]
            z = ccz[csl]
            ok = (x >= 0) & (x < R) & (y >= 0) & (y < R) & (z >= 0) & (z < R)
            ln = x * (R * R) + y * R + z
            own = ok & (ln >= lo) & (ln < hi)
            ln = jnp.where(own, ln, dump)
            usl = slice(u * 16, u * 16 + 16)
            lin128[usl] = ln
            val128[usl] = g * 128 + u * 16 + lax.iota(jnp.int32, 16) + 1
        pltpu.async_copy(val128, shr.at[lin128], sem).wait()

    plsc.subcore_barrier()

    g0 = wid * RPW
    for d, src_c in enumerate((ctx, cty, ctz)):
        pltpu.sync_copy(src_c.at[pl.ds(g0, RPW)], cg[d])

    @pl.loop(0, K)
    def _(k):
        dx = k // 9 - 1
        dy = (k // 3) % 3 - 1
        dz = k % 3 - 1
        for j in range(20):
            csl = pl.ds(j * 16, 16)
            x = cg[0][csl] + dx
            y = cg[1][csl] + dy
            z = cg[2][csl] + dz
            ok = (x >= 0) & (x < R) & (y >= 0) & (y < R) & (z >= 0) & (z < R)
            ln = x * (R * R) + y * R + z
            ln = jnp.where(ok, ln, G_DUMP)
            nl5[j // 4][(j % 4) * 16:(j % 4) * 16 + 16] = ln
        ds_ = [pltpu.async_copy(shr.at[nl5[r]], tv5[r], sem) for r in range(5)]
        for d_ in ds_:
            d_.wait()
        for j in range(20):
            csl = pl.ds(j * 16, 16)
            t = tv5[j // 4][(j % 4) * 16:(j % 4) * 16 + 16]
            zr = N + ((wid * 67 + k * 37 + j * 16 + lax.iota(jnp.int32, 16)) & 127)
            gvv[csl] = jnp.where(t > 0, t - 1, zr)
        pltpu.sync_copy(gvv, gidx.at[pl.ds((wid * K + k) * RPW, RPW)])


@functools.partial(
    pl.kernel,
    out_type=jax.ShapeDtypeStruct((K * NP, C), jnp.float32),
    mesh=_sc_mesh,
    scratch_types=(
        [pltpu.VMEM((K * RPW,), jnp.int32)]
        + [pltpu.VMEM((RPW, C), jnp.float32) for _ in range(2)]
        + [pltpu.SemaphoreType.DMA for _ in range(20)]
    ),
)
def _sc_gather(h, gidx, gout, idxall, b0, b1, *sems):
    gsems = (sems[0:5], sems[5:10])
    ssems = (sems[10:15], sems[15:20])
    cid = lax.axis_index("c")
    sid = lax.axis_index("s")
    wid = sid * NC + cid
    base = wid * RPW
    bufs = (b0, b1)
    pltpu.sync_copy(gidx.at[pl.ds(wid * (K * RPW), K * RPW)], idxall)
    gath = [[None] * 5, [None] * 5]
    stor = [[None] * 5, [None] * 5]
    for k in range(K + 1):
        b = k % 2
        if k < K:
            for r in range(5):
                if stor[b][r] is not None:
                    stor[b][r].wait()
                gath[b][r] = pltpu.async_copy(
                    h.at[idxall.at[pl.ds(k * RPW + r * 64, 64)]],
                    bufs[b].at[pl.ds(r * 64, 64)],
                    gsems[b][r],
                )
        pb = (k + 1) % 2
        if k >= 1:
            for r in range(5):
                gath[pb][r].wait()
                stor[pb][r] = pltpu.async_copy(
                    bufs[pb].at[pl.ds(r * 64, 64)],
                    gout.at[pl.ds((k - 1) * NP + base + r * 64, 64)],
                    ssems[pb][r],
                )
    for r in range(5):
        stor[(K - 1) % 2][r].wait()


def _ln(y):
    mu = jnp.mean(y, axis=-1, keepdims=True)
    var = jnp.mean((y - mu) ** 2, axis=-1, keepdims=True)
    return (y - mu) / jnp.sqrt(var + EPS)


def _mask_rows(y, rowbase):
    rid = rowbase + lax.broadcasted_iota(jnp.int32, (BLK, 1), 0)
    return jnp.where(rid < N, y, 0.0)


def _conv_acc(g_ref, w_ref):
    acc = jnp.zeros((BLK, C), jnp.float32)
    for k in range(K):
        acc = acc + jnp.dot(g_ref[k], w_ref[k], preferred_element_type=jnp.float32)
    return acc


def _tc_pre(x_ref, g_ref, b_ref, o_ref):
    rowbase = pl.program_id(0) * BLK
    y = _ln(x_ref[...]) * g_ref[...] + b_ref[...]
    y = y * jax.nn.sigmoid(y)
    o_ref[...] = _mask_rows(y, rowbase)


def _tc_mid(g_ref, w_ref, b_ref, o_ref):
    rowbase = pl.program_id(0) * BLK
    y = _ln(_conv_acc(g_ref, w_ref) + b_ref[...])
    y = y * jax.nn.sigmoid(y)
    o_ref[...] = _mask_rows(y, rowbase)


def _tc_post(g_ref, w_ref, b_ref, f_ref, o_ref):
    o_ref[...] = _conv_acc(g_ref, w_ref) + b_ref[...] + f_ref[...]


_vec_spec = pl.BlockSpec((1, C), lambda i: (0, 0))
_row_spec = pl.BlockSpec((BLK, C), lambda i: (i, 0))
_g_spec = pl.BlockSpec((K, BLK, C), lambda i: (0, i, 0))
_w_spec = pl.BlockSpec((K, C, C), lambda i: (0, 0, 0))

_pre_call = pl.pallas_call(
    _tc_pre,
    grid=(NBLK,),
    in_specs=[_row_spec, _vec_spec, _vec_spec],
    out_specs=_row_spec,
    out_shape=jax.ShapeDtypeStruct((NP, C), jnp.float32),
)

_mid_call = pl.pallas_call(
    _tc_mid,
    grid=(NBLK,),
    in_specs=[_g_spec, _w_spec, _vec_spec],
    out_specs=_row_spec,
    out_shape=jax.ShapeDtypeStruct((NP, C), jnp.float32),
)

_post_call = pl.pallas_call(
    _tc_post,
    grid=(NBLK,),
    in_specs=[_g_spec, _w_spec, _vec_spec, _row_spec],
    out_specs=_row_spec,
    out_shape=jax.ShapeDtypeStruct((NP, C), jnp.float32),
)


def kernel(feats, coords, gamma, beta, W1, b1, W2, b2):
    f32 = jnp.float32
    fp = jnp.zeros((NP, C), f32).at[:N].set(feats.astype(f32))
    cp = jnp.full((NP, 3), R, jnp.int32).at[:N].set(coords.astype(jnp.int32))
    cx, cy, cz = cp[:, 0].copy(), cp[:, 1].copy(), cp[:, 2].copy()

    gidx = _sc_index(cx, cy, cz)

    h1 = _pre_call(fp, gamma.reshape(1, C), beta.reshape(1, C))
    G1 = _sc_gather(h1, gidx).reshape(K, NP, C)
    h2 = _mid_call(G1, W1, b1.reshape(1, C))
    G2 = _sc_gather(h2, gidx).reshape(K, NP, C)
    out = _post_call(G2, W2, b2.reshape(1, C), fp)
    return out[:N]
